# baseline (speedup 1.0000x reference)
import functools

import jax
import jax.numpy as jnp
from jax import lax
from jax.experimental import pallas as pl
from jax.experimental.pallas import tpu as pltpu
from jax.experimental.pallas import tpu_sc as plsc

_TEMP = 1.0
_LBW = 0.01
_BT = 256
_NC, _NS, _L = 2, 16, 16
_WREP = 128
_NW = _NC * _NS


def _router_meta_body(nb, x_ref, rw_ref, pos_ref, w_ref, be_ref, act_ref,
                      chg_ref, slot_ref, aux_ref):
    t, _ = x_ref.shape
    e = rw_ref.shape[0]
    x = x_ref[...]
    rw = rw_ref[...]
    logits = lax.dot_general(x, rw, (((1,), (1,)), ((), ())),
                             preferred_element_type=jnp.float32)
    logits = logits / _TEMP
    m = jnp.max(logits, axis=-1, keepdims=True)
    ex = jnp.exp(logits - m)
    probs = ex / jnp.sum(ex, axis=-1, keepdims=True)
    pmax = jnp.max(probs, axis=-1, keepdims=True)
    eids = lax.broadcasted_iota(jnp.int32, probs.shape, 1)
    idx = jnp.min(jnp.where(probs == pmax, eids, e), axis=-1, keepdims=True)
    oh = (eids == idx).astype(jnp.float32)

    c = oh
    k = 1
    while k < t:
        c = c + jnp.concatenate(
            [jnp.zeros((k, e), jnp.float32), c[:t - k]], axis=0)
        k *= 2
    counts = c[t - 1:t, :]
    rank = jnp.sum(c * oh, axis=-1, keepdims=True) - 1.0

    ac = jnp.ceil(counts / _BT) * _BT
    co = ac
    k = 1
    while k < e:
        co = co + jnp.concatenate(
            [jnp.zeros((1, k), jnp.float32), co[:, :e - k]], axis=1)
        k *= 2
    offs_incl = co
    offs_excl = offs_incl - ac

    pos = jnp.sum(oh * offs_excl, axis=-1, keepdims=True) + rank
    pos_ref[...] = pos.astype(jnp.int32)
    w_ref[...] = jnp.broadcast_to(pmax, (t, _WREP))

    total = offs_incl[:, e - 1:e]
    sb = lax.broadcasted_iota(jnp.int32, (nb, e), 0).astype(jnp.float32) * _BT
    be_raw = jnp.sum((sb >= offs_incl).astype(jnp.int32), axis=-1,
                     keepdims=True)
    be_last = jnp.sum(((total - _BT) >= offs_incl).astype(jnp.int32), axis=-1,
                      keepdims=True)
    bec = jnp.minimum(be_raw, be_last)
    be_ref[...] = bec
    act_ref[...] = (sb[:, :1] < total).astype(jnp.int32)

    prev = jnp.concatenate(
        [jnp.full((1, 1), -1, jnp.int32), bec[:nb - 1]], axis=0)
    chg = (bec != prev).astype(jnp.float32)
    trf = chg
    k = 1
    while k < nb:
        trf = trf + jnp.concatenate(
            [jnp.zeros((k, 1), jnp.float32), trf[:nb - k]], axis=0)
        k *= 2
    slot = trf - 3.0 * jnp.floor(trf / 3.0)
    chg_ref[...] = chg.astype(jnp.int32)
    slot_ref[...] = slot.astype(jnp.int32)

    mean_probs = jnp.mean(probs, axis=0, keepdims=True)
    freq = counts / float(t)
    aux_ref[...] = _LBW * float(e) * jnp.sum(mean_probs * freq,
                                             axis=-1, keepdims=True)


def _router_meta(x_flat, router_w, nb):
    t = x_flat.shape[0]
    return pl.pallas_call(
        functools.partial(_router_meta_body, nb),
        out_shape=[
            jax.ShapeDtypeStruct((t, 1), jnp.int32),
            jax.ShapeDtypeStruct((t, _WREP), jnp.float32),
            jax.ShapeDtypeStruct((nb, 1), jnp.int32),
            jax.ShapeDtypeStruct((nb, 1), jnp.int32),
            jax.ShapeDtypeStruct((nb, 1), jnp.int32),
            jax.ShapeDtypeStruct((nb, 1), jnp.int32),
            jax.ShapeDtypeStruct((1, 1), jnp.float32),
        ],
    )(x_flat, router_w)


def _dispatch_sc(x_flat, pos, w_rep, p):
    t, hid = x_flat.shape
    tok_per = t // _NW
    mesh = plsc.VectorSubcoreMesh(core_axis_name="c", subcore_axis_name="s")

    @functools.partial(
        pl.kernel,
        mesh=mesh,
        compiler_params=pltpu.CompilerParams(needs_layout_passes=False),
        out_type=[
            jax.ShapeDtypeStruct((p, hid), jnp.float32),
            jax.ShapeDtypeStruct((p, _WREP), jnp.float32),
        ],
        scratch_types=[
            pltpu.VMEM((tok_per,), jnp.int32),
            pltpu.VMEM((tok_per, hid), jnp.float32),
            pltpu.VMEM((tok_per, _WREP), jnp.float32),
            pltpu.SemaphoreType.DMA,
            pltpu.SemaphoreType.DMA,
            pltpu.SemaphoreType.DMA,
        ],
    )
    def dispatch(x_hbm, pos_hbm, wrep_hbm, xbuf_hbm, wbuf_hbm,
                 idx_v, rows_v, wrep_v, sem_x, sem_w, sem_p):
        wid = lax.axis_index("s") * _NC + lax.axis_index("c")
        base = wid * tok_per
        cp = pltpu.async_copy(pos_hbm.at[pl.ds(base, tok_per)], idx_v, sem_p)
        cr = pltpu.async_copy(x_hbm.at[pl.ds(base, tok_per)], rows_v, sem_x)
        cv = pltpu.async_copy(wrep_hbm.at[pl.ds(base, tok_per)], wrep_v, sem_w)
        cp.wait()
        cr.wait()
        cx = pltpu.async_copy(rows_v, xbuf_hbm.at[idx_v], sem_x)
        cv.wait()
        cw = pltpu.async_copy(wrep_v, wbuf_hbm.at[idx_v], sem_w)
        cx.wait()
        cw.wait()

    return dispatch(x_flat, pos, w_rep)


def _ffn_body(nb, be_sm, act_sm, chg_sm, slot_sm, x_ref, w1_hbm, b1_ref,
              w2_hbm, b2_ref, wtok_ref, y_ref, w1_scr, w2_scr, w1b_scr,
              w2b_scr, sem1, sem2):
    b = pl.program_id(0)

    def w_copy(bb):
        s = slot_sm[bb, 0]
        eidx = be_sm[bb, 0]
        c1 = pltpu.make_async_copy(w1_hbm.at[eidx], w1_scr.at[s], sem1.at[s])
        c2 = pltpu.make_async_copy(w2_hbm.at[eidx], w2_scr.at[s], sem2.at[s])
        return c1, c2

    def issue(bb):
        @pl.when(chg_sm[bb, 0] == 1)
        def _():
            c1, c2 = w_copy(bb)
            c1.start()
            c2.start()

    @pl.when(b == 0)
    def _():
        issue(0)
        issue(1)
        issue(2)

    @pl.when((b > 0) & (b + 2 < nb))
    def _():
        issue(b + 2)

    @pl.when(chg_sm[b, 0] == 1)
    def _():
        c1, c2 = w_copy(b)
        c1.wait()
        c2.wait()
        s = slot_sm[b, 0]
        w1b_scr[pl.ds(s, 1)] = w1_scr[pl.ds(s, 1)].astype(jnp.bfloat16)
        w2b_scr[pl.ds(s, 1)] = w2_scr[pl.ds(s, 1)].astype(jnp.bfloat16)

    @pl.when(act_sm[b, 0] == 1)
    def _():
        s = slot_sm[b, 0]
        w1 = w1b_scr[pl.ds(s, 1)][0]
        w2 = w2b_scr[pl.ds(s, 1)][0]
        x = x_ref[...].astype(jnp.bfloat16)
        h = lax.dot_general(x, w1, (((1,), (1,)), ((), ())),
                            preferred_element_type=jnp.float32)
        h = h + b1_ref[0]
        h = 0.5 * h * (1.0 + lax.erf(h * (2.0 ** -0.5)))
        y = lax.dot_general(h.astype(jnp.bfloat16), w2,
                            (((1,), (1,)), ((), ())),
                            preferred_element_type=jnp.float32)
        y = y + b2_ref[0]
        y_ref[...] = y * wtok_ref[:, :1]


def _ffn_tc(x_buf, w_buf, fc1_w, fc1_b, fc2_w, fc2_b, be, act, chg, slot):
    p, hid = x_buf.shape
    e, ffn, _ = fc1_w.shape
    nb = p // _BT
    grid_spec = pltpu.PrefetchScalarGridSpec(
        num_scalar_prefetch=4,
        grid=(nb,),
        in_specs=[
            pl.BlockSpec((_BT, hid), lambda b, *_: (b, 0)),
            pl.BlockSpec(memory_space=pl.ANY),
            pl.BlockSpec((1, 1, ffn), lambda b, be, *_: (be[b, 0], 0, 0)),
            pl.BlockSpec(memory_space=pl.ANY),
            pl.BlockSpec((1, 1, hid), lambda b, be, *_: (be[b, 0], 0, 0)),
            pl.BlockSpec((_BT, _WREP), lambda b, *_: (b, 0)),
        ],
        out_specs=pl.BlockSpec((_BT, hid), lambda b, *_: (b, 0)),
        scratch_shapes=[
            pltpu.VMEM((3, ffn, hid), jnp.float32),
            pltpu.VMEM((3, hid, ffn), jnp.float32),
            pltpu.VMEM((3, ffn, hid), jnp.bfloat16),
            pltpu.VMEM((3, hid, ffn), jnp.bfloat16),
            pltpu.SemaphoreType.DMA((3,)),
            pltpu.SemaphoreType.DMA((3,)),
        ],
    )
    return pl.pallas_call(
        functools.partial(_ffn_body, nb),
        grid_spec=grid_spec,
        out_shape=jax.ShapeDtypeStruct((p, hid), jnp.float32),
    )(be, act, chg, slot, x_buf, fc1_w, fc1_b.reshape(e, 1, ffn), fc2_w,
      fc2_b.reshape(e, 1, hid), w_buf)


def _combine_sc(y_buf, pos, t):
    p, hid = y_buf.shape
    tok_per = t // _NW
    mesh = plsc.VectorSubcoreMesh(core_axis_name="c", subcore_axis_name="s")

    @functools.partial(
        pl.kernel,
        mesh=mesh,
        compiler_params=pltpu.CompilerParams(needs_layout_passes=False),
        out_type=jax.ShapeDtypeStruct((t, hid), jnp.float32),
        scratch_types=[
            pltpu.VMEM((tok_per,), jnp.int32),
            pltpu.VMEM((tok_per, hid), jnp.float32),
            pltpu.SemaphoreType.DMA,
        ],
    )
    def combine(ybuf_hbm, pos_hbm, out_hbm, idx_v, rows_v, sem):
        wid = lax.axis_index("s") * _NC + lax.axis_index("c")
        base = wid * tok_per
        pltpu.sync_copy(pos_hbm.at[pl.ds(base, tok_per)], idx_v)
        pltpu.async_copy(ybuf_hbm.at[idx_v], rows_v, sem).wait()
        pltpu.sync_copy(rows_v, out_hbm.at[pl.ds(base, tok_per)])

    return combine(y_buf, pos)


def kernel(x, router_w, fc1_w, fc1_b, fc2_w, fc2_b):
    b, s, d = x.shape
    t = b * s
    e = router_w.shape[0]
    p = t + e * _BT
    nb = p // _BT

    x_flat = x.reshape(t, d)
    pos2d, w2d, be2d, act2d, chg2d, slot2d, aux2d = _router_meta(
        x_flat, router_w, nb)
    pos = pos2d.reshape(t)
    x_buf, w_buf = _dispatch_sc(x_flat, pos, w2d, p)
    y_buf = _ffn_tc(x_buf, w_buf, fc1_w, fc1_b, fc2_w, fc2_b, be2d, act2d,
                    chg2d, slot2d)
    out_flat = _combine_sc(y_buf, pos, t)
    return out_flat.reshape(b, s, d), aux2d[0, 0]


# speedup vs baseline: 1.0713x; 1.0713x over previous
import functools

import jax
import jax.numpy as jnp
from jax import lax
from jax.experimental import pallas as pl
from jax.experimental.pallas import tpu as pltpu
from jax.experimental.pallas import tpu_sc as plsc

_TEMP = 1.0
_LBW = 0.01
_BT = 256
_NC, _NS, _L = 2, 16, 16
_WREP = 128
_NW = _NC * _NS


def _router_meta_body(nb, x_ref, rw_ref, pos_ref, w_ref, be_ref, act_ref,
                      chg_ref, slot_ref, xi_ref, yi_ref, aux_ref):
    t, _ = x_ref.shape
    e = rw_ref.shape[0]
    x = x_ref[...]
    rw = rw_ref[...]
    logits = lax.dot_general(x, rw, (((1,), (1,)), ((), ())),
                             preferred_element_type=jnp.float32)
    logits = logits / _TEMP
    m = jnp.max(logits, axis=-1, keepdims=True)
    ex = jnp.exp(logits - m)
    probs = ex / jnp.sum(ex, axis=-1, keepdims=True)
    pmax = jnp.max(probs, axis=-1, keepdims=True)
    eids = lax.broadcasted_iota(jnp.int32, probs.shape, 1)
    idx = jnp.min(jnp.where(probs == pmax, eids, e), axis=-1, keepdims=True)
    oh = (eids == idx).astype(jnp.float32)

    c = oh
    k = 1
    while k < t:
        c = c + jnp.concatenate(
            [jnp.zeros((k, e), jnp.float32), c[:t - k]], axis=0)
        k *= 2
    counts = c[t - 1:t, :]
    rank = jnp.sum(c * oh, axis=-1, keepdims=True) - 1.0

    ac = jnp.ceil(counts / _BT) * _BT
    co = ac
    k = 1
    while k < e:
        co = co + jnp.concatenate(
            [jnp.zeros((1, k), jnp.float32), co[:, :e - k]], axis=1)
        k *= 2
    offs_incl = co
    offs_excl = offs_incl - ac

    pos = jnp.sum(oh * offs_excl, axis=-1, keepdims=True) + rank
    pos_ref[...] = pos.astype(jnp.int32)
    w_ref[...] = jnp.broadcast_to(pmax, (t, _WREP))

    total = offs_incl[:, e - 1:e]
    sb = lax.broadcasted_iota(jnp.int32, (nb, e), 0).astype(jnp.float32) * _BT
    be_raw = jnp.sum((sb >= offs_incl).astype(jnp.int32), axis=-1,
                     keepdims=True)
    be_last = jnp.sum(((total - _BT) >= offs_incl).astype(jnp.int32), axis=-1,
                      keepdims=True)
    bec = jnp.minimum(be_raw, be_last)
    be_ref[...] = bec
    act_ref[...] = (sb[:, :1] < total).astype(jnp.int32)

    prev = jnp.concatenate(
        [jnp.full((1, 1), -1, jnp.int32), bec[:nb - 1]], axis=0)
    chg = (bec != prev).astype(jnp.float32)
    trf = chg
    k = 1
    while k < nb:
        trf = trf + jnp.concatenate(
            [jnp.zeros((k, 1), jnp.float32), trf[:nb - k]], axis=0)
        k *= 2
    slot = trf - 3.0 * jnp.floor(trf / 3.0)
    chg_ref[...] = chg.astype(jnp.int32)
    slot_ref[...] = slot.astype(jnp.int32)

    la = (total / _BT).astype(jnp.int32) - 1
    bidx = lax.broadcasted_iota(jnp.int32, (nb, 1), 0)
    xi_ref[...] = jnp.minimum(bidx, la)
    yi_ref[...] = jnp.where(bidx <= la, bidx, nb)

    mean_probs = jnp.mean(probs, axis=0, keepdims=True)
    freq = counts / float(t)
    aux_ref[...] = _LBW * float(e) * jnp.sum(mean_probs * freq,
                                             axis=-1, keepdims=True)


def _router_meta(x_flat, router_w, nb):
    t = x_flat.shape[0]
    return pl.pallas_call(
        functools.partial(_router_meta_body, nb),
        out_shape=[
            jax.ShapeDtypeStruct((t, 1), jnp.int32),
            jax.ShapeDtypeStruct((t, _WREP), jnp.float32),
            jax.ShapeDtypeStruct((nb, 1), jnp.int32),
            jax.ShapeDtypeStruct((nb, 1), jnp.int32),
            jax.ShapeDtypeStruct((nb, 1), jnp.int32),
            jax.ShapeDtypeStruct((nb, 1), jnp.int32),
            jax.ShapeDtypeStruct((nb, 1), jnp.int32),
            jax.ShapeDtypeStruct((nb, 1), jnp.int32),
            jax.ShapeDtypeStruct((1, 1), jnp.float32),
        ],
    )(x_flat, router_w)


def _dispatch_sc(x_flat, pos, w_rep, p):
    t, hid = x_flat.shape
    tok_per = t // _NW
    mesh = plsc.VectorSubcoreMesh(core_axis_name="c", subcore_axis_name="s")

    @functools.partial(
        pl.kernel,
        mesh=mesh,
        compiler_params=pltpu.CompilerParams(needs_layout_passes=False),
        out_type=[
            jax.ShapeDtypeStruct((p, hid), jnp.float32),
            jax.ShapeDtypeStruct((p, _WREP), jnp.float32),
        ],
        scratch_types=[
            pltpu.VMEM((tok_per,), jnp.int32),
            pltpu.VMEM((tok_per, hid), jnp.float32),
            pltpu.VMEM((tok_per, _WREP), jnp.float32),
            pltpu.SemaphoreType.DMA,
            pltpu.SemaphoreType.DMA,
            pltpu.SemaphoreType.DMA,
        ],
    )
    def dispatch(x_hbm, pos_hbm, wrep_hbm, xbuf_hbm, wbuf_hbm,
                 idx_v, rows_v, wrep_v, sem_x, sem_w, sem_p):
        wid = lax.axis_index("s") * _NC + lax.axis_index("c")
        base = wid * tok_per
        cp = pltpu.async_copy(pos_hbm.at[pl.ds(base, tok_per)], idx_v, sem_p)
        cr = pltpu.async_copy(x_hbm.at[pl.ds(base, tok_per)], rows_v, sem_x)
        cv = pltpu.async_copy(wrep_hbm.at[pl.ds(base, tok_per)], wrep_v, sem_w)
        cp.wait()
        cr.wait()
        cx = pltpu.async_copy(rows_v, xbuf_hbm.at[idx_v], sem_x)
        cv.wait()
        cw = pltpu.async_copy(wrep_v, wbuf_hbm.at[idx_v], sem_w)
        cx.wait()
        cw.wait()

    return dispatch(x_flat, pos, w_rep)


def _ffn_body(nb, be_sm, act_sm, chg_sm, slot_sm, xi_sm, yi_sm, x_ref, w1_hbm,
              b1_ref, w2_hbm, b2_ref, wtok_ref, y_ref, w1_scr, w2_scr,
              w1b_scr, w2b_scr, sem1, sem2):
    b = pl.program_id(0)

    def w_copy(bb):
        s = slot_sm[bb, 0]
        eidx = be_sm[bb, 0]
        c1 = pltpu.make_async_copy(w1_hbm.at[eidx], w1_scr.at[s], sem1.at[s])
        c2 = pltpu.make_async_copy(w2_hbm.at[eidx], w2_scr.at[s], sem2.at[s])
        return c1, c2

    def issue(bb):
        @pl.when(chg_sm[bb, 0] == 1)
        def _():
            c1, c2 = w_copy(bb)
            c1.start()
            c2.start()

    @pl.when(b == 0)
    def _():
        issue(0)
        issue(1)
        issue(2)

    @pl.when((b > 0) & (b + 2 < nb))
    def _():
        issue(b + 2)

    @pl.when(chg_sm[b, 0] == 1)
    def _():
        c1, c2 = w_copy(b)
        c1.wait()
        c2.wait()
        s = slot_sm[b, 0]
        w1b_scr[pl.ds(s, 1)] = w1_scr[pl.ds(s, 1)].astype(jnp.bfloat16)
        w2b_scr[pl.ds(s, 1)] = w2_scr[pl.ds(s, 1)].astype(jnp.bfloat16)

    @pl.when(act_sm[b, 0] == 1)
    def _():
        s = slot_sm[b, 0]
        w1 = w1b_scr[pl.ds(s, 1)][0]
        w2 = w2b_scr[pl.ds(s, 1)][0]
        x = x_ref[...].astype(jnp.bfloat16)
        h = lax.dot_general(x, w1, (((1,), (1,)), ((), ())),
                            preferred_element_type=jnp.float32)
        h = h + b1_ref[0]
        h = 0.5 * h * (1.0 + lax.erf(h * (2.0 ** -0.5)))
        y = lax.dot_general(h.astype(jnp.bfloat16), w2,
                            (((1,), (1,)), ((), ())),
                            preferred_element_type=jnp.float32)
        y = y + b2_ref[0]
        y_ref[...] = y * wtok_ref[:, :1]


def _ffn_tc(x_buf, w_buf, fc1_w, fc1_b, fc2_w, fc2_b, be, act, chg, slot,
            xi, yi):
    p, hid = x_buf.shape
    e, ffn, _ = fc1_w.shape
    nb = p // _BT
    grid_spec = pltpu.PrefetchScalarGridSpec(
        num_scalar_prefetch=6,
        grid=(nb,),
        in_specs=[
            pl.BlockSpec((_BT, hid),
                         lambda b, be, act, chg, slot, xi, yi: (xi[b, 0], 0)),
            pl.BlockSpec(memory_space=pl.ANY),
            pl.BlockSpec((1, 1, ffn),
                         lambda b, be, *_: (be[b, 0], 0, 0)),
            pl.BlockSpec(memory_space=pl.ANY),
            pl.BlockSpec((1, 1, hid),
                         lambda b, be, *_: (be[b, 0], 0, 0)),
            pl.BlockSpec((_BT, _WREP),
                         lambda b, be, act, chg, slot, xi, yi: (xi[b, 0], 0)),
        ],
        out_specs=pl.BlockSpec(
            (_BT, hid), lambda b, be, act, chg, slot, xi, yi: (yi[b, 0], 0)),
        scratch_shapes=[
            pltpu.VMEM((3, ffn, hid), jnp.float32),
            pltpu.VMEM((3, hid, ffn), jnp.float32),
            pltpu.VMEM((3, ffn, hid), jnp.bfloat16),
            pltpu.VMEM((3, hid, ffn), jnp.bfloat16),
            pltpu.SemaphoreType.DMA((3,)),
            pltpu.SemaphoreType.DMA((3,)),
        ],
    )
    return pl.pallas_call(
        functools.partial(_ffn_body, nb),
        grid_spec=grid_spec,
        out_shape=jax.ShapeDtypeStruct((p + _BT, hid), jnp.float32),
    )(be, act, chg, slot, xi, yi, x_buf, fc1_w, fc1_b.reshape(e, 1, ffn),
      fc2_w, fc2_b.reshape(e, 1, hid), w_buf)


def _combine_sc(y_buf, pos, t):
    p, hid = y_buf.shape
    tok_per = t // _NW
    mesh = plsc.VectorSubcoreMesh(core_axis_name="c", subcore_axis_name="s")

    @functools.partial(
        pl.kernel,
        mesh=mesh,
        compiler_params=pltpu.CompilerParams(needs_layout_passes=False),
        out_type=jax.ShapeDtypeStruct((t, hid), jnp.float32),
        scratch_types=[
            pltpu.VMEM((tok_per,), jnp.int32),
            pltpu.VMEM((tok_per, hid), jnp.float32),
            pltpu.SemaphoreType.DMA,
        ],
    )
    def combine(ybuf_hbm, pos_hbm, out_hbm, idx_v, rows_v, sem):
        wid = lax.axis_index("s") * _NC + lax.axis_index("c")
        base = wid * tok_per
        pltpu.sync_copy(pos_hbm.at[pl.ds(base, tok_per)], idx_v)
        pltpu.async_copy(ybuf_hbm.at[idx_v], rows_v, sem).wait()
        pltpu.sync_copy(rows_v, out_hbm.at[pl.ds(base, tok_per)])

    return combine(y_buf, pos)


def kernel(x, router_w, fc1_w, fc1_b, fc2_w, fc2_b):
    b, s, d = x.shape
    t = b * s
    e = router_w.shape[0]
    p = t + e * _BT
    nb = p // _BT

    x_flat = x.reshape(t, d)
    (pos2d, w2d, be2d, act2d, chg2d, slot2d, xi2d, yi2d,
     aux2d) = _router_meta(x_flat, router_w, nb)
    pos = pos2d.reshape(t)
    x_buf, w_buf = _dispatch_sc(x_flat, pos, w2d, p)
    y_buf = _ffn_tc(x_buf, w_buf, fc1_w, fc1_b, fc2_w, fc2_b, be2d, act2d,
                    chg2d, slot2d, xi2d, yi2d)
    out_flat = _combine_sc(y_buf, pos, t)
    return out_flat.reshape(b, s, d), aux2d[0, 0]
